# baseline (speedup 1.0000x reference)
import functools
import math

import jax
import jax.numpy as jnp
from jax import lax
from jax.experimental import pallas as pl
from jax.experimental.pallas import tpu as pltpu
from jax.experimental.pallas import tpu_sc as plsc

D_MODEL = 64
N_ROWS = 4096
N_SEQ = 200
NUM_CORES = 2
NUM_SUBCORES = 16
NUM_WORKERS = NUM_CORES * NUM_SUBCORES
CBLK = N_ROWS // 128
UNITS_TOTAL = N_SEQ * CBLK
UPW = UNITS_TOTAL // NUM_WORKERS
NBUF = 4
SCALE = math.sqrt(D_MODEL)

_mesh = plsc.VectorSubcoreMesh(core_axis_name="c", subcore_axis_name="s")

N_TOKEN = 1000000
NTILE_T = 7813
LUT2_ROWS = NTILE_T * 64


@functools.partial(
    pl.kernel,
    mesh=_mesh,
    out_type=jax.ShapeDtypeStruct((LUT2_ROWS, 128), jnp.float32),
    scratch_types=[
        pltpu.VMEM((2, 64, 129), jnp.float32),
        pltpu.VMEM((2, 64, 128), jnp.float32),
        pltpu.SemaphoreType.DMA((2,)),
        pltpu.SemaphoreType.DMA((2,)),
    ],
    compiler_params=pltpu.CompilerParams(needs_layout_passes=False),
)
def _relayout_sc(lutT_hbm, lut2_hbm, src_v, dst_v, rsem, wsem):
    wid = lax.axis_index("s") * NUM_CORES + lax.axis_index("c")
    lanes = lax.iota(jnp.int32, 16)
    perm = [(lanes + k) & 15 for k in range(16)]
    lanes2 = lanes * 2
    n = jnp.where(wid < NTILE_T - (NTILE_T // NUM_WORKERS) * NUM_WORKERS,
                  NTILE_T // NUM_WORKERS + 1, NTILE_T // NUM_WORKERS)

    def fire_src(k, b):
        c2 = wid + k * NUM_WORKERS
        pltpu.async_copy(
            lutT_hbm.at[pl.ds(0, 64), pl.ds(c2 * 128, 128)],
            src_v.at[b, pl.ds(0, 64), pl.ds(0, 128)], rsem.at[b])

    def wait_src(b):
        pltpu.make_async_copy(
            lutT_hbm.at[pl.ds(0, 64), pl.ds(0, 128)],
            src_v.at[b, pl.ds(0, 64), pl.ds(0, 128)], rsem.at[b]).wait()

    def fire_store(k, b):
        c2 = wid + k * NUM_WORKERS
        pltpu.async_copy(dst_v.at[b], lut2_hbm.at[pl.ds(c2 * 64, 64)],
                         wsem.at[b])

    def wait_store(b):
        pltpu.make_async_copy(dst_v.at[b], lut2_hbm.at[pl.ds(0, 64)],
                              wsem.at[b]).wait()

    fire_src(0, 0)

    def pair_body(kk, carry):
        for b in range(2):
            k2 = kk * 2 + b

            @pl.when(k2 < n)
            def _():
                wait_src(b)

                @pl.when(k2 + 1 < n)
                def _():
                    fire_src(k2 + 1, 1 - b)

                @pl.when(k2 >= 2)
                def _():
                    wait_store(b)

                @plsc.parallel_loop(0, 32, unroll=4)
                def _(blk):
                    a0 = lax.shift_right_logical(blk, 3) * 16
                    c0 = (blk & 7) * 16
                    c0m = (blk & 3) * 16
                    h = lax.shift_right_logical(blk, 2) & 1
                    colv = lanes2 + (a0 * 2 + h)
                    rowa = lanes + a0
                    for kd in range(16):
                        vals = plsc.load_gather(
                            src_v.at[b], [perm[kd] + c0m, colv])
                        plsc.store_scatter(
                            dst_v.at[b], [rowa, perm[kd] + c0], vals)

                fire_store(k2, b)
        return carry

    lax.fori_loop(0, (NTILE_T // NUM_WORKERS + 2) // 2, pair_body, 0)
    wait_store(0)
    wait_store(1)


@functools.partial(
    pl.kernel,
    mesh=_mesh,
    out_type=jax.ShapeDtypeStruct((N_SEQ, D_MODEL, N_ROWS), jnp.float32),
    scratch_types=[
        pltpu.VMEM((NBUF, 128), jnp.int32),
        pltpu.VMEM((NBUF, 128), jnp.int32),
        pltpu.VMEM((NBUF, 128, 128), jnp.float32),
        pltpu.VMEM((2, D_MODEL, 128), jnp.float32),
        pltpu.VMEM((128,), jnp.int32),
        pltpu.SemaphoreType.DMA((NBUF,)),
        pltpu.SemaphoreType.DMA((NBUF,)),
        pltpu.SemaphoreType.DMA((2,)),
    ],
    compiler_params=pltpu.CompilerParams(needs_layout_passes=False),
)
def _embed_sc(xt_hbm, lut2_hbm, out_hbm, idx_v, idx2_v, rows_v, outt_v,
              par_v, isem, gsem, ssem):
    wid = lax.axis_index("s") * NUM_CORES + lax.axis_index("c")
    ubase = wid * UPW
    lanes = lax.iota(jnp.int32, 16)
    perm = [(lanes + k) & 15 for k in range(16)]

    def unit_sc(k):
        u = ubase + k
        return u >> 5, u & (CBLK - 1)

    def fire_idx(k, b):
        s, c = unit_sc(k)
        pltpu.async_copy(xt_hbm.at[s, pl.ds(c * 128, 128)],
                         idx_v.at[b], isem.at[b])

    def wait_idx(b):
        s, c = unit_sc(0)
        pltpu.make_async_copy(xt_hbm.at[s, pl.ds(0, 128)],
                              idx_v.at[b], isem.at[b]).wait()

    def fire_gather(b):
        pltpu.async_copy(lut2_hbm.at[idx2_v.at[b]], rows_v.at[b], gsem.at[b])

    def wait_gather(b):
        pltpu.make_async_copy(lut2_hbm.at[idx2_v.at[b]],
                              rows_v.at[b], gsem.at[b]).wait()

    def compute_idx2(b):
        for g in range(8):
            t = idx_v[b, pl.ds(g * 16, 16)]
            idx2_v[b, pl.ds(g * 16, 16)] = lax.shift_right_logical(t, 1)

    def stage(k, b):
        wait_idx(b)
        compute_idx2(b)
        fire_gather(b)

    def fire_stores(k, ob):
        s, c = unit_sc(k)
        pltpu.async_copy(
            outt_v.at[ob],
            out_hbm.at[s, pl.ds(0, D_MODEL), pl.ds(c * 128, 128)],
            ssem.at[ob])

    def wait_stores(ob):
        s, c = unit_sc(0)
        pltpu.make_async_copy(
            outt_v.at[ob],
            out_hbm.at[s, pl.ds(0, D_MODEL), pl.ds(0, 128)],
            ssem.at[ob]).wait()

    fire_idx(0, 0)
    fire_idx(1, 1)
    stage(0, 0)
    fire_idx(2, 2)
    stage(1, 1)
    fire_idx(3, 3)

    def quad_body(kk, carry):
        for b in range(NBUF):
            k = kk * NBUF + b
            for g in range(8):
                t = idx_v[b, pl.ds(g * 16, 16)]
                par_v[pl.ds(g * 16, 16)] = (t & 1) * D_MODEL

            @pl.when(k + 2 < UPW)
            def _():
                stage(k + 2, (b + 2) % NBUF)

            @pl.when(k + NBUF < UPW)
            def _():
                fire_idx(k + NBUF, b)

            @pl.when(k >= 2)
            def _():
                wait_stores(b % 2)

            wait_gather(b)

            @plsc.parallel_loop(0, 32, unroll=2)
            def _(blk):
                i0 = lax.shift_right_logical(blk, 2) * 16
                c0 = (blk & 3) * 16
                rowi = lanes + i0
                par = par_v[pl.ds(i0, 16)]
                for k in range(16):
                    colr = (perm[k] + c0) + par
                    vals = plsc.load_gather(rows_v.at[b], [rowi, colr])
                    plsc.store_scatter(
                        outt_v.at[b % 2], [perm[k] + c0, rowi], vals * SCALE)

            fire_stores(k, b % 2)
        return carry

    lax.fori_loop(0, UPW // NBUF, quad_body, 0)
    wait_stores(0)
    wait_stores(1)


def kernel(x, lut):
    lut2 = _relayout_sc(lut.T)
    out = _embed_sc(x.T, lut2)
    return jnp.transpose(out, (2, 0, 1))
